# baseline (speedup 1.0000x reference)
import jax
import jax.numpy as jnp
from jax import lax
from jax.experimental import pallas as pl
from jax.experimental.pallas import tpu as pltpu
from jax.experimental.pallas import tpu_sc as plsc

N = 10000
E = 320000
D = 128
L = 16
NC = 2
NS = 16
NW = NC * NS
EPW = E // NW
CH = 32
NCHUNK = 315
NB = 3
OUTER = NCHUNK // NB
NPAD = N + 8
RPT = 624
TAIL0 = NS * RPT
TAILR = N - TAIL0

ROWBLK = 2000
NBLK = N // ROWBLK


def _sc_edge_body(tbl_hbm, comb_hbm, wrow_hbm,
                  out_hbm,
                  comb, sidx, gbuf, wbuf, red,
                  sem_g, sem_s, sem_i):
    cid = lax.axis_index("c")
    sid = lax.axis_index("s")
    wid = sid * NC + cid

    def zero_row(r, _):
        for j in range(D // L):
            gbuf[0, r, pl.ds(j * L, L)] = jnp.zeros((L,), jnp.float32)
        return 0
    lax.fori_loop(0, CH, zero_row, 0)
    row0 = sid * RPT
    for k in range(RPT // CH):
        pltpu.sync_copy(gbuf.at[0, pl.ds(0, CH)],
                        red.at[pl.ds(row0 + k * CH, CH)])
    zrem = RPT - (RPT // CH) * CH
    pltpu.sync_copy(gbuf.at[0, pl.ds(0, zrem)],
                    red.at[pl.ds(row0 + RPT - zrem, zrem)])

    @pl.when(sid == 0)
    def _zero_tail():
        pltpu.sync_copy(gbuf.at[0, pl.ds(0, TAILR)],
                        red.at[pl.ds(TAIL0, TAILR)])

    pltpu.sync_copy(wrow_hbm, wbuf)
    plsc.subcore_barrier()

    wv = [wbuf[pl.ds(j * L, L)] for j in range(D // L)]

    pltpu.sync_copy(comb_hbm.at[wid, 0], comb.at[0])
    pltpu.sync_copy(comb_hbm.at[wid, 1], comb.at[1])
    pltpu.async_copy(comb_hbm.at[wid, 2], comb.at[2], sem_i.at[2])
    for b in range(2):
        pltpu.async_copy(tbl_hbm.at[comb.at[b, 0]], gbuf.at[b], sem_g.at[b])

    def outer_body(p, _):
        for b0 in range(NB):
            c = p * NB + b0
            b2 = (b0 + 2) % NB
            pltpu.make_async_copy(tbl_hbm.at[pl.ds(0, 2 * CH)], gbuf.at[b0],
                                  sem_g.at[b0]).wait()

            def group_body(g, _):
                gsl = pl.ds(g * L, L)
                sidx[b0, gsl] = comb[b0, 1, gsl]
                tv = lax.bitcast_convert_type(
                    comb[b0, 1, pl.ds(CH + g * L, L)], jnp.float32)
                for u in range(L):
                    e = g * L + u
                    t = tv[u]
                    for j in range(D // L):
                        sl = pl.ds(j * L, L)
                        x = gbuf[b0, e, sl] + gbuf[b0, CH + e, sl] + t * wv[j]
                        gbuf[b0, e, sl] = jnp.where(
                            x >= 0, x, jnp.float32(0.01) * x)
                return 0
            lax.fori_loop(0, CH // L, group_body, 0)

            pltpu.async_copy(gbuf.at[b0, pl.ds(0, CH)], red.at[sidx.at[b0]],
                             sem_s.at[b0], add=True)

            @pl.when(c + 2 < NCHUNK)
            def _issue_gathers():
                @pl.when(c >= 1)
                def _drain_scatter():
                    pltpu.make_async_copy(tbl_hbm.at[pl.ds(0, CH)],
                                          gbuf.at[b2, pl.ds(0, CH)],
                                          sem_s.at[b2]).wait()
                pltpu.make_async_copy(comb_hbm.at[wid, 0], comb.at[b2],
                                      sem_i.at[b2]).wait()
                pltpu.async_copy(tbl_hbm.at[comb.at[b2, 0]], gbuf.at[b2],
                                 sem_g.at[b2])

            @pl.when(c + 3 < NCHUNK)
            def _prefetch_idx():
                pltpu.async_copy(comb_hbm.at[wid, c + 3], comb.at[b0],
                                 sem_i.at[b0])
        return 0

    lax.fori_loop(0, OUTER, outer_body, 0)

    for c in range(NCHUNK - 3, NCHUNK):
        pltpu.make_async_copy(tbl_hbm.at[pl.ds(0, CH)],
                              gbuf.at[c % NB, pl.ds(0, CH)],
                              sem_s.at[c % NB]).wait()

    plsc.subcore_barrier()
    pltpu.sync_copy(red.at[pl.ds(row0, RPT)],
                    out_hbm.at[cid, pl.ds(row0, RPT)])

    @pl.when(sid == 0)
    def _copy_tail():
        pltpu.sync_copy(red.at[pl.ds(TAIL0, TAILR)],
                        out_hbm.at[cid, pl.ds(TAIL0, TAILR)])


_sc_edge = pl.kernel(
    _sc_edge_body,
    out_type=jax.ShapeDtypeStruct((NC, N, D), jnp.float32),
    mesh=plsc.VectorSubcoreMesh(core_axis_name="c", subcore_axis_name="s"),
    scratch_types=[
        pltpu.VMEM((NB, 2, 2 * CH), jnp.int32),
        pltpu.VMEM((NB, CH), jnp.int32),
        pltpu.VMEM((NB, 2 * CH, D), jnp.float32),
        pltpu.VMEM((D,), jnp.float32),
        pltpu.VMEM_SHARED((NPAD, D), jnp.float32),
        pltpu.SemaphoreType.DMA((NB,)),
        pltpu.SemaphoreType.DMA((NB,)),
        pltpu.SemaphoreType.DMA((NB,)),
    ],
)


def _leaky(x):
    return jnp.where(x >= 0, x, 0.01 * x)


def _tc_proj_body(x_ref, wa_ref, wb_ref, be_ref, tbl_ref):
    k = pl.program_id(0)
    x = x_ref[...]
    w = jnp.where(k == 0, wa_ref[...], wb_ref[...])
    bias = jnp.where(k == 0, jnp.float32(0), jnp.float32(1)) * be_ref[...]
    tbl_ref[...] = jnp.dot(x, w, preferred_element_type=jnp.float32) + bias


def _tc_proj(x, wa, wb, be):
    row = pl.BlockSpec((ROWBLK, D), lambda k, i: (i, 0))
    full = pl.BlockSpec((D, D), lambda k, i: (0, 0))
    vec = pl.BlockSpec((1, D), lambda k, i: (0, 0))
    trow = pl.BlockSpec((ROWBLK, D), lambda k, i: (k * NBLK + i, 0))
    return pl.pallas_call(
        _tc_proj_body,
        grid=(2, NBLK),
        in_specs=[row, full, full, vec],
        out_specs=trow,
        out_shape=jax.ShapeDtypeStruct((2 * N, D), jnp.float32),
    )(x, wa, wb, be)


def _tc_node_proj_body(x_ref, r0_ref, r1_ref, wna_ref, wnb_ref, bn_ref,
                       wa_ref, wb_ref, be_ref, h_ref, tbl_ref):
    k = pl.program_id(0)
    x = x_ref[...]
    red = r0_ref[...] + r1_ref[...]
    h = _leaky(jnp.dot(x, wna_ref[...], preferred_element_type=jnp.float32)
               + jnp.dot(red, wnb_ref[...], preferred_element_type=jnp.float32)
               + bn_ref[...])
    h_ref[...] = h
    w = jnp.where(k == 0, wa_ref[...], wb_ref[...])
    bias = jnp.where(k == 0, jnp.float32(0), jnp.float32(1)) * be_ref[...]
    tbl_ref[...] = jnp.dot(h, w, preferred_element_type=jnp.float32) + bias


def _tc_node_proj(x, r0, r1, wna, wnb, bn, wa, wb, be):
    row = pl.BlockSpec((ROWBLK, D), lambda k, i: (i, 0))
    full = pl.BlockSpec((D, D), lambda k, i: (0, 0))
    vec = pl.BlockSpec((1, D), lambda k, i: (0, 0))
    trow = pl.BlockSpec((ROWBLK, D), lambda k, i: (k * NBLK + i, 0))
    return pl.pallas_call(
        _tc_node_proj_body,
        grid=(2, NBLK),
        in_specs=[row, row, row, full, full, vec, full, full, vec],
        out_specs=[row, trow],
        out_shape=[jax.ShapeDtypeStruct((N, D), jnp.float32),
                   jax.ShapeDtypeStruct((2 * N, D), jnp.float32)],
    )(x, r0, r1, wna, wnb, bn, wa, wb, be)


def _tc_node_body(x_ref, r0_ref, r1_ref, wna_ref, wnb_ref, bn_ref, h_ref):
    x = x_ref[...]
    red = r0_ref[...] + r1_ref[...]
    h_ref[...] = _leaky(
        jnp.dot(x, wna_ref[...], preferred_element_type=jnp.float32)
        + jnp.dot(red, wnb_ref[...], preferred_element_type=jnp.float32)
        + bn_ref[...])


def _tc_node(x, r0, r1, wna, wnb, bn):
    row = pl.BlockSpec((ROWBLK, D), lambda i: (i, 0))
    full = pl.BlockSpec((D, D), lambda i: (0, 0))
    vec = pl.BlockSpec((1, D), lambda i: (0, 0))
    return pl.pallas_call(
        _tc_node_body,
        grid=(NBLK,),
        in_specs=[row, row, row, full, full, vec],
        out_specs=row,
        out_shape=jax.ShapeDtypeStruct((N, D), jnp.float32),
    )(x, r0, r1, wna, wnb, bn)


def _pack_edges(src, dst, traj):
    pw = NCHUNK * CH - EPW
    src3 = jnp.concatenate(
        [src.reshape(NW, EPW), jnp.zeros((NW, pw), jnp.int32)], axis=1)
    gdst3 = jnp.concatenate(
        [dst.reshape(NW, EPW) + N, jnp.zeros((NW, pw), jnp.int32)], axis=1)
    sdst3 = jnp.concatenate(
        [dst.reshape(NW, EPW), jnp.full((NW, pw), N, jnp.int32)], axis=1)
    tbits = lax.bitcast_convert_type(traj, jnp.int32)
    traj3 = jnp.concatenate(
        [tbits.reshape(NW, EPW), jnp.zeros((NW, pw), jnp.int32)], axis=1)
    ch0 = jnp.concatenate([src3.reshape(NW, NCHUNK, CH),
                           gdst3.reshape(NW, NCHUNK, CH)], axis=2)
    ch1 = jnp.concatenate([sdst3.reshape(NW, NCHUNK, CH),
                           traj3.reshape(NW, NCHUNK, CH)], axis=2)
    return jnp.stack([ch0, ch1], axis=2)


def kernel(nf, edge_index, traj, We0, be0, Wn0, bn0, We1, be1, Wn1, bn1):
    comb = _pack_edges(edge_index[0], edge_index[1], traj)

    tbl0 = _tc_proj(nf, We0[:D], We0[D:2 * D], be0[None, :])
    parts0 = _sc_edge(tbl0, comb, We0[2 * D])
    h, tbl1 = _tc_node_proj(nf, parts0[0], parts0[1],
                            Wn0[:D], Wn0[D:], bn0[None, :],
                            We1[:D], We1[D:2 * D], be1[None, :])
    parts1 = _sc_edge(tbl1, comb, We1[2 * D])
    out = _tc_node(h, parts1[0], parts1[1], Wn1[:D], Wn1[D:], bn1[None, :])
    return out


# speedup vs baseline: 1.3187x; 1.3187x over previous
import jax
import jax.numpy as jnp
from jax import lax
from jax.experimental import pallas as pl
from jax.experimental.pallas import tpu as pltpu
from jax.experimental.pallas import tpu_sc as plsc

N = 10000
E = 320000
D = 128
L = 16
NC = 2
NS = 16
NW = NC * NS
EPW = E // NW
CH = 32
NCHUNK = 315
NB = 3
OUTER = NCHUNK // NB
NPAD = N + 8
RPT = 624
TAIL0 = NS * RPT
TAILR = N - TAIL0

ROWBLK = 2000
NBLK = N // ROWBLK


def _sc_edge_body(ps_hbm, pd_hbm, comb_hbm, wrow_hbm,
                  out_hbm,
                  comb, sidx, buf_a, buf_b, wbuf, red,
                  sem_a, sem_b, sem_s, sem_i):
    cid = lax.axis_index("c")
    sid = lax.axis_index("s")
    wid = sid * NC + cid

    def zero_row(r, _):
        for j in range(D // L):
            buf_a[0, r, pl.ds(j * L, L)] = jnp.zeros((L,), jnp.float32)
        return 0
    lax.fori_loop(0, CH, zero_row, 0)
    row0 = sid * RPT
    for k in range(RPT // CH):
        pltpu.sync_copy(buf_a.at[0], red.at[pl.ds(row0 + k * CH, CH)])
    zrem = RPT - (RPT // CH) * CH
    pltpu.sync_copy(buf_a.at[0, pl.ds(0, zrem)],
                    red.at[pl.ds(row0 + RPT - zrem, zrem)])

    @pl.when(sid == 0)
    def _zero_tail():
        pltpu.sync_copy(buf_a.at[0, pl.ds(0, TAILR)],
                        red.at[pl.ds(TAIL0, TAILR)])

    pltpu.sync_copy(wrow_hbm, wbuf)
    plsc.subcore_barrier()

    wv = [wbuf[pl.ds(j * L, L)] for j in range(D // L)]

    pltpu.sync_copy(comb_hbm.at[wid, 0], comb.at[0])
    pltpu.sync_copy(comb_hbm.at[wid, 1], comb.at[1])
    pltpu.async_copy(comb_hbm.at[wid, 2], comb.at[2], sem_i.at[2])
    for b in range(2):
        for hh in range(CH // 16):
            hs = pl.ds(hh * 16, 16)
            pltpu.async_copy(ps_hbm.at[comb.at[b, 0, hs]],
                             buf_a.at[b, hs], sem_a.at[b])
            pltpu.async_copy(pd_hbm.at[comb.at[b, 1, hs]],
                             buf_b.at[b, hs], sem_b.at[b])

    def outer_body(p, _):
        for b0 in range(NB):
            c = p * NB + b0
            b2 = (b0 + 2) % NB
            pltpu.make_async_copy(ps_hbm.at[pl.ds(0, CH)], buf_a.at[b0],
                                  sem_a.at[b0]).wait()
            pltpu.make_async_copy(pd_hbm.at[pl.ds(0, CH)], buf_b.at[b0],
                                  sem_b.at[b0]).wait()

            def group_body(g, _):
                gsl = pl.ds(g * L, L)
                sidx[b0, gsl] = comb[b0, 1, gsl]
                tv = lax.bitcast_convert_type(comb[b0, 2, gsl], jnp.float32)
                for u in range(L):
                    e = g * L + u
                    t = tv[u]
                    for j in range(D // L):
                        sl = pl.ds(j * L, L)
                        x = buf_a[b0, e, sl] + buf_b[b0, e, sl] + t * wv[j]
                        buf_a[b0, e, sl] = jnp.where(
                            x >= 0, x, jnp.float32(0.01) * x)
                return 0
            lax.fori_loop(0, CH // L, group_body, 0)

            pltpu.async_copy(buf_a.at[b0], red.at[sidx.at[b0]],
                             sem_s.at[b0], add=True)

            @pl.when(c + 2 < NCHUNK)
            def _issue_gathers():
                @pl.when(c >= 1)
                def _drain_scatter():
                    pltpu.make_async_copy(ps_hbm.at[pl.ds(0, CH)],
                                          buf_a.at[b2], sem_s.at[b2]).wait()
                pltpu.make_async_copy(comb_hbm.at[wid, 0], comb.at[b2],
                                      sem_i.at[b2]).wait()
                for hh in range(CH // 16):
                    hs = pl.ds(hh * 16, 16)
                    pltpu.async_copy(ps_hbm.at[comb.at[b2, 0, hs]],
                                     buf_a.at[b2, hs], sem_a.at[b2])
                    pltpu.async_copy(pd_hbm.at[comb.at[b2, 1, hs]],
                                     buf_b.at[b2, hs], sem_b.at[b2])

            @pl.when(c + 3 < NCHUNK)
            def _prefetch_idx():
                pltpu.async_copy(comb_hbm.at[wid, c + 3], comb.at[b0],
                                 sem_i.at[b0])
        return 0

    lax.fori_loop(0, OUTER, outer_body, 0)

    for c in range(NCHUNK - 3, NCHUNK):
        pltpu.make_async_copy(ps_hbm.at[pl.ds(0, CH)], buf_a.at[c % NB],
                              sem_s.at[c % NB]).wait()

    plsc.subcore_barrier()
    pltpu.sync_copy(red.at[pl.ds(row0, RPT)],
                    out_hbm.at[cid, pl.ds(row0, RPT)])

    @pl.when(sid == 0)
    def _copy_tail():
        pltpu.sync_copy(red.at[pl.ds(TAIL0, TAILR)],
                        out_hbm.at[cid, pl.ds(TAIL0, TAILR)])


_sc_edge = pl.kernel(
    _sc_edge_body,
    out_type=jax.ShapeDtypeStruct((NC, N, D), jnp.float32),
    mesh=plsc.VectorSubcoreMesh(core_axis_name="c", subcore_axis_name="s"),
    scratch_types=[
        pltpu.VMEM((NB, 3, CH), jnp.int32),
        pltpu.VMEM((NB, CH), jnp.int32),
        pltpu.VMEM((NB, CH, D), jnp.float32),
        pltpu.VMEM((NB, CH, D), jnp.float32),
        pltpu.VMEM((D,), jnp.float32),
        pltpu.VMEM_SHARED((NPAD, D), jnp.float32),
        pltpu.SemaphoreType.DMA((NB,)),
        pltpu.SemaphoreType.DMA((NB,)),
        pltpu.SemaphoreType.DMA((NB,)),
        pltpu.SemaphoreType.DMA((NB,)),
    ],
)


def _leaky(x):
    return jnp.where(x >= 0, x, 0.01 * x)


def _pack_bf16_pairs(x):
    xb = x.astype(jnp.bfloat16)
    return lax.bitcast_convert_type(
        xb.reshape(x.shape[0], D // 2, 2), jnp.int32)


def _tc_proj_body(x_ref, wa_ref, wb_ref, be_ref, ps_ref, pd_ref):
    x = x_ref[...]
    ps_ref[...] = jnp.dot(x, wa_ref[...], preferred_element_type=jnp.float32)
    pd_ref[...] = jnp.dot(x, wb_ref[...],
                          preferred_element_type=jnp.float32) + be_ref[...]


def _tc_proj(x, wa, wb, be):
    row = pl.BlockSpec((ROWBLK, D), lambda i: (i, 0))
    full = pl.BlockSpec((D, D), lambda i: (0, 0))
    vec = pl.BlockSpec((1, D), lambda i: (0, 0))
    return pl.pallas_call(
        _tc_proj_body,
        grid=(NBLK,),
        in_specs=[row, full, full, vec],
        out_specs=[row, row],
        out_shape=[jax.ShapeDtypeStruct((N, D), jnp.float32),
                   jax.ShapeDtypeStruct((N, D), jnp.float32)],
    )(x, wa, wb, be)


def _tc_node_proj_body(x_ref, r0_ref, r1_ref, wna_ref, wnb_ref, bn_ref,
                       wa_ref, wb_ref, be_ref, h_ref, ps_ref, pd_ref):
    x = x_ref[...]
    red = r0_ref[...] + r1_ref[...]
    h = _leaky(jnp.dot(x, wna_ref[...], preferred_element_type=jnp.float32)
               + jnp.dot(red, wnb_ref[...], preferred_element_type=jnp.float32)
               + bn_ref[...])
    h_ref[...] = h
    ps_ref[...] = jnp.dot(h, wa_ref[...], preferred_element_type=jnp.float32)
    pd_ref[...] = jnp.dot(h, wb_ref[...],
                          preferred_element_type=jnp.float32) + be_ref[...]


def _tc_node_proj(x, r0, r1, wna, wnb, bn, wa, wb, be):
    row = pl.BlockSpec((ROWBLK, D), lambda i: (i, 0))
    full = pl.BlockSpec((D, D), lambda i: (0, 0))
    vec = pl.BlockSpec((1, D), lambda i: (0, 0))
    return pl.pallas_call(
        _tc_node_proj_body,
        grid=(NBLK,),
        in_specs=[row, row, row, full, full, vec, full, full, vec],
        out_specs=[row, row, row],
        out_shape=[jax.ShapeDtypeStruct((N, D), jnp.float32),
                   jax.ShapeDtypeStruct((N, D), jnp.float32),
                   jax.ShapeDtypeStruct((N, D), jnp.float32)],
    )(x, r0, r1, wna, wnb, bn, wa, wb, be)


def _tc_node_body(x_ref, r0_ref, r1_ref, wna_ref, wnb_ref, bn_ref, h_ref):
    x = x_ref[...]
    red = r0_ref[...] + r1_ref[...]
    h_ref[...] = _leaky(
        jnp.dot(x, wna_ref[...], preferred_element_type=jnp.float32)
        + jnp.dot(red, wnb_ref[...], preferred_element_type=jnp.float32)
        + bn_ref[...])


def _tc_node(x, r0, r1, wna, wnb, bn):
    row = pl.BlockSpec((ROWBLK, D), lambda i: (i, 0))
    full = pl.BlockSpec((D, D), lambda i: (0, 0))
    vec = pl.BlockSpec((1, D), lambda i: (0, 0))
    return pl.pallas_call(
        _tc_node_body,
        grid=(NBLK,),
        in_specs=[row, row, row, full, full, vec],
        out_specs=row,
        out_shape=jax.ShapeDtypeStruct((N, D), jnp.float32),
    )(x, r0, r1, wna, wnb, bn)


def _pack_edges(src, dst, traj):
    pw = NCHUNK * CH - EPW
    src3 = jnp.concatenate(
        [src.reshape(NW, EPW), jnp.zeros((NW, pw), jnp.int32)], axis=1)
    dst3 = jnp.concatenate(
        [dst.reshape(NW, EPW), jnp.full((NW, pw), N, jnp.int32)], axis=1)
    tbits = lax.bitcast_convert_type(traj, jnp.int32)
    traj3 = jnp.concatenate(
        [tbits.reshape(NW, EPW), jnp.zeros((NW, pw), jnp.int32)], axis=1)
    return jnp.stack([src3.reshape(NW, NCHUNK, CH),
                      dst3.reshape(NW, NCHUNK, CH),
                      traj3.reshape(NW, NCHUNK, CH)], axis=2)


def kernel(nf, edge_index, traj, We0, be0, Wn0, bn0, We1, be1, Wn1, bn1):
    comb = _pack_edges(edge_index[0], edge_index[1], traj)

    ps0, pd0 = _tc_proj(nf, We0[:D], We0[D:2 * D], be0[None, :])
    parts0 = _sc_edge(ps0, pd0, comb, We0[2 * D])
    h, ps1, pd1 = _tc_node_proj(nf, parts0[0], parts0[1],
                                Wn0[:D], Wn0[D:], bn0[None, :],
                                We1[:D], We1[D:2 * D], be1[None, :])
    parts1 = _sc_edge(ps1, pd1, comb, We1[2 * D])
    out = _tc_node(h, parts1[0], parts1[1], Wn1[:D], Wn1[D:], bn1[None, :])
    return out


# speedup vs baseline: 1.3296x; 1.0083x over previous
import jax
import jax.numpy as jnp
from jax import lax
from jax.experimental import pallas as pl
from jax.experimental.pallas import tpu as pltpu
from jax.experimental.pallas import tpu_sc as plsc

N = 10000
E = 320000
D = 128
L = 16
NC = 2
NS = 16
NW = NC * NS
EPW = E // NW
CH = 32
NCHUNK = 315
NB = 3
OUTER = NCHUNK // NB
NPAD = N + 8
RPT = 624
TAIL0 = NS * RPT
TAILR = N - TAIL0

ROWBLK = 2000
NBLK = N // ROWBLK


def _sc_edge_body(ps_hbm, pd_hbm, comb_hbm, wrow_hbm,
                  out_hbm,
                  comb, sidx, buf_a, buf_b, wbuf, red,
                  sem_a, sem_b, sem_s, sem_i):
    cid = lax.axis_index("c")
    sid = lax.axis_index("s")
    wid = sid * NC + cid

    pltpu.sync_copy(comb_hbm.at[wid, 0], comb.at[0])
    pltpu.sync_copy(comb_hbm.at[wid, 1], comb.at[1])
    pltpu.async_copy(comb_hbm.at[wid, 2], comb.at[2], sem_i.at[2])
    for b in range(2):
        pltpu.async_copy(ps_hbm.at[comb.at[b, 0]], buf_a.at[b], sem_a.at[b])
        pltpu.async_copy(pd_hbm.at[comb.at[b, 1]], buf_b.at[b], sem_b.at[b])

    def zero_row(r, _):
        for j in range(D // L):
            buf_a[2, r, pl.ds(j * L, L)] = jnp.zeros((L,), jnp.float32)
        return 0
    lax.fori_loop(0, CH, zero_row, 0)
    row0 = sid * RPT
    for k in range(RPT // CH):
        pltpu.sync_copy(buf_a.at[2], red.at[pl.ds(row0 + k * CH, CH)])
    zrem = RPT - (RPT // CH) * CH
    pltpu.sync_copy(buf_a.at[2, pl.ds(0, zrem)],
                    red.at[pl.ds(row0 + RPT - zrem, zrem)])

    @pl.when(sid == 0)
    def _zero_tail():
        pltpu.sync_copy(buf_a.at[2, pl.ds(0, TAILR)],
                        red.at[pl.ds(TAIL0, TAILR)])

    pltpu.sync_copy(wrow_hbm, wbuf)
    plsc.subcore_barrier()

    wv = [wbuf[pl.ds(j * L, L)] for j in range(D // L)]

    def outer_body(p, _):
        for b0 in range(NB):
            c = p * NB + b0
            b2 = (b0 + 2) % NB
            pltpu.make_async_copy(ps_hbm.at[pl.ds(0, CH)], buf_a.at[b0],
                                  sem_a.at[b0]).wait()
            pltpu.make_async_copy(pd_hbm.at[pl.ds(0, CH)], buf_b.at[b0],
                                  sem_b.at[b0]).wait()

            def group_body(g, _):
                gsl = pl.ds(g * L, L)
                sidx[b0, gsl] = comb[b0, 1, gsl]
                tv = lax.bitcast_convert_type(comb[b0, 2, gsl], jnp.float32)
                for u in range(L):
                    e = g * L + u
                    t = tv[u]
                    for j in range(D // L):
                        sl = pl.ds(j * L, L)
                        x = buf_a[b0, e, sl] + buf_b[b0, e, sl] + t * wv[j]
                        buf_a[b0, e, sl] = jnp.where(
                            x >= 0, x, jnp.float32(0.01) * x)
                return 0
            lax.fori_loop(0, CH // L, group_body, 0)

            pltpu.async_copy(buf_a.at[b0], red.at[sidx.at[b0]],
                             sem_s.at[b0], add=True)

            @pl.when(c + 2 < NCHUNK)
            def _issue_gathers():
                @pl.when(c >= 1)
                def _drain_scatter():
                    pltpu.make_async_copy(ps_hbm.at[pl.ds(0, CH)],
                                          buf_a.at[b2], sem_s.at[b2]).wait()
                pltpu.make_async_copy(comb_hbm.at[wid, 0], comb.at[b2],
                                      sem_i.at[b2]).wait()
                pltpu.async_copy(ps_hbm.at[comb.at[b2, 0]], buf_a.at[b2],
                                 sem_a.at[b2])
                pltpu.async_copy(pd_hbm.at[comb.at[b2, 1]], buf_b.at[b2],
                                 sem_b.at[b2])

            @pl.when(c + 3 < NCHUNK)
            def _prefetch_idx():
                pltpu.async_copy(comb_hbm.at[wid, c + 3], comb.at[b0],
                                 sem_i.at[b0])
        return 0

    lax.fori_loop(0, OUTER, outer_body, 0)

    for c in range(NCHUNK - 3, NCHUNK):
        pltpu.make_async_copy(ps_hbm.at[pl.ds(0, CH)], buf_a.at[c % NB],
                              sem_s.at[c % NB]).wait()

    plsc.subcore_barrier()
    pltpu.sync_copy(red.at[pl.ds(row0, RPT)],
                    out_hbm.at[cid, pl.ds(row0, RPT)])

    @pl.when(sid == 0)
    def _copy_tail():
        pltpu.sync_copy(red.at[pl.ds(TAIL0, TAILR)],
                        out_hbm.at[cid, pl.ds(TAIL0, TAILR)])


_sc_edge = pl.kernel(
    _sc_edge_body,
    out_type=jax.ShapeDtypeStruct((NC, N, D), jnp.float32),
    mesh=plsc.VectorSubcoreMesh(core_axis_name="c", subcore_axis_name="s"),
    scratch_types=[
        pltpu.VMEM((NB, 3, CH), jnp.int32),
        pltpu.VMEM((NB, CH), jnp.int32),
        pltpu.VMEM((NB, CH, D), jnp.float32),
        pltpu.VMEM((NB, CH, D), jnp.float32),
        pltpu.VMEM((D,), jnp.float32),
        pltpu.VMEM_SHARED((NPAD, D), jnp.float32),
        pltpu.SemaphoreType.DMA((NB,)),
        pltpu.SemaphoreType.DMA((NB,)),
        pltpu.SemaphoreType.DMA((NB,)),
        pltpu.SemaphoreType.DMA((NB,)),
    ],
)


def _leaky(x):
    return jnp.where(x >= 0, x, 0.01 * x)


def _pack_bf16_pairs(x):
    xb = x.astype(jnp.bfloat16)
    return lax.bitcast_convert_type(
        xb.reshape(x.shape[0], D // 2, 2), jnp.int32)


def _tc_proj_body(x_ref, wa_ref, wb_ref, be_ref, ps_ref, pd_ref):
    x = x_ref[...]
    ps_ref[...] = jnp.dot(x, wa_ref[...], preferred_element_type=jnp.float32)
    pd_ref[...] = jnp.dot(x, wb_ref[...],
                          preferred_element_type=jnp.float32) + be_ref[...]


def _tc_proj(x, wa, wb, be):
    row = pl.BlockSpec((ROWBLK, D), lambda i: (i, 0))
    full = pl.BlockSpec((D, D), lambda i: (0, 0))
    vec = pl.BlockSpec((1, D), lambda i: (0, 0))
    return pl.pallas_call(
        _tc_proj_body,
        grid=(NBLK,),
        in_specs=[row, full, full, vec],
        out_specs=[row, row],
        out_shape=[jax.ShapeDtypeStruct((N, D), jnp.float32),
                   jax.ShapeDtypeStruct((N, D), jnp.float32)],
    )(x, wa, wb, be)


def _tc_node_proj_body(x_ref, r0_ref, r1_ref, wna_ref, wnb_ref, bn_ref,
                       wa_ref, wb_ref, be_ref, h_ref, ps_ref, pd_ref):
    x = x_ref[...]
    red = r0_ref[...] + r1_ref[...]
    h = _leaky(jnp.dot(x, wna_ref[...], preferred_element_type=jnp.float32)
               + jnp.dot(red, wnb_ref[...], preferred_element_type=jnp.float32)
               + bn_ref[...])
    h_ref[...] = h
    ps_ref[...] = jnp.dot(h, wa_ref[...], preferred_element_type=jnp.float32)
    pd_ref[...] = jnp.dot(h, wb_ref[...],
                          preferred_element_type=jnp.float32) + be_ref[...]


def _tc_node_proj(x, r0, r1, wna, wnb, bn, wa, wb, be):
    row = pl.BlockSpec((ROWBLK, D), lambda i: (i, 0))
    full = pl.BlockSpec((D, D), lambda i: (0, 0))
    vec = pl.BlockSpec((1, D), lambda i: (0, 0))
    return pl.pallas_call(
        _tc_node_proj_body,
        grid=(NBLK,),
        in_specs=[row, row, row, full, full, vec, full, full, vec],
        out_specs=[row, row, row],
        out_shape=[jax.ShapeDtypeStruct((N, D), jnp.float32),
                   jax.ShapeDtypeStruct((N, D), jnp.float32),
                   jax.ShapeDtypeStruct((N, D), jnp.float32)],
    )(x, r0, r1, wna, wnb, bn, wa, wb, be)


def _tc_node_body(x_ref, r0_ref, r1_ref, wna_ref, wnb_ref, bn_ref, h_ref):
    x = x_ref[...]
    red = r0_ref[...] + r1_ref[...]
    h_ref[...] = _leaky(
        jnp.dot(x, wna_ref[...], preferred_element_type=jnp.float32)
        + jnp.dot(red, wnb_ref[...], preferred_element_type=jnp.float32)
        + bn_ref[...])


def _tc_node(x, r0, r1, wna, wnb, bn):
    row = pl.BlockSpec((ROWBLK, D), lambda i: (i, 0))
    full = pl.BlockSpec((D, D), lambda i: (0, 0))
    vec = pl.BlockSpec((1, D), lambda i: (0, 0))
    return pl.pallas_call(
        _tc_node_body,
        grid=(NBLK,),
        in_specs=[row, row, row, full, full, vec],
        out_specs=row,
        out_shape=jax.ShapeDtypeStruct((N, D), jnp.float32),
    )(x, r0, r1, wna, wnb, bn)


def _pack_edges(src, dst, traj):
    pw = NCHUNK * CH - EPW
    src3 = jnp.concatenate(
        [src.reshape(NW, EPW), jnp.zeros((NW, pw), jnp.int32)], axis=1)
    dst3 = jnp.concatenate(
        [dst.reshape(NW, EPW), jnp.full((NW, pw), N, jnp.int32)], axis=1)
    tbits = lax.bitcast_convert_type(traj, jnp.int32)
    traj3 = jnp.concatenate(
        [tbits.reshape(NW, EPW), jnp.zeros((NW, pw), jnp.int32)], axis=1)
    return jnp.stack([src3.reshape(NW, NCHUNK, CH),
                      dst3.reshape(NW, NCHUNK, CH),
                      traj3.reshape(NW, NCHUNK, CH)], axis=2)


def kernel(nf, edge_index, traj, We0, be0, Wn0, bn0, We1, be1, Wn1, bn1):
    comb = _pack_edges(edge_index[0], edge_index[1], traj)

    ps0, pd0 = _tc_proj(nf, We0[:D], We0[D:2 * D], be0[None, :])
    parts0 = _sc_edge(ps0, pd0, comb, We0[2 * D])
    h, ps1, pd1 = _tc_node_proj(nf, parts0[0], parts0[1],
                                Wn0[:D], Wn0[D:], bn0[None, :],
                                We1[:D], We1[D:2 * D], be1[None, :])
    parts1 = _sc_edge(ps1, pd1, comb, We1[2 * D])
    out = _tc_node(h, parts1[0], parts1[1], Wn1[:D], Wn1[D:], bn1[None, :])
    return out
